# baseline (speedup 1.0000x reference)
import functools

import jax
import jax.numpy as jnp
from jax.experimental import pallas as pl
from jax.experimental.pallas import tpu as pltpu


def _se_body(x_ref, w_ref, b_ref, o_ref, *, inv_hw):
    x = x_ref[...]
    pooled = jnp.sum(x, axis=1, keepdims=True) * inv_hw
    logits = jax.lax.dot_general(
        w_ref[...], pooled, (((1,), (0,)), ((), ())),
        preferred_element_type=jnp.float32,
    ) + b_ref[...]
    gate = jax.nn.sigmoid(logits)
    o_ref[...] = x * gate


def kernel(x, weight, bias):
    B, C, H, W = x.shape
    HW = H * W
    x2 = x.reshape(B * C, HW)
    w = jnp.asarray(weight).reshape(C, C).astype(jnp.float32)
    b_col = jnp.asarray(bias).reshape(C, 1).astype(jnp.float32)

    out = pl.pallas_call(
        functools.partial(_se_body, inv_hw=1.0 / HW),
        out_shape=jax.ShapeDtypeStruct((B * C, HW), x.dtype),
        grid=(B,),
        in_specs=[
            pl.BlockSpec((C, HW), lambda b: (b, 0)),
            pl.BlockSpec((C, C), lambda b: (0, 0)),
            pl.BlockSpec((C, 1), lambda b: (0, 0)),
        ],
        out_specs=pl.BlockSpec((C, HW), lambda b: (b, 0)),
        compiler_params=pltpu.CompilerParams(
            dimension_semantics=("parallel",),
            vmem_limit_bytes=48 << 20,
        ),
    )(x2, w, b_col)
    return out.reshape(B, C, H, W)


# speedup vs baseline: 2.2194x; 2.2194x over previous
import functools

import jax
import jax.numpy as jnp
from jax.experimental import pallas as pl
from jax.experimental.pallas import tpu as pltpu


def _se_body(x_ref, w_ref, b_ref, o_ref, *, inv_hw):
    x = x_ref[0]
    pooled = jnp.sum(x, axis=1, keepdims=True) * inv_hw
    logits = jax.lax.dot_general(
        w_ref[...], pooled, (((1,), (0,)), ((), ())),
        preferred_element_type=jnp.float32,
    ) + b_ref[...]
    gate = jax.nn.sigmoid(logits)
    o_ref[0] = x * gate


def kernel(x, weight, bias):
    B, C, H, W = x.shape
    HW = H * W
    x3 = x.reshape(B, C, HW)
    w = jnp.asarray(weight).reshape(C, C).astype(jnp.float32)
    b_col = jnp.asarray(bias).reshape(C, 1).astype(jnp.float32)

    out = pl.pallas_call(
        functools.partial(_se_body, inv_hw=1.0 / HW),
        out_shape=jax.ShapeDtypeStruct((B, C, HW), x.dtype),
        grid=(B,),
        in_specs=[
            pl.BlockSpec((1, C, HW), lambda b: (b, 0, 0)),
            pl.BlockSpec((C, C), lambda b: (0, 0)),
            pl.BlockSpec((C, 1), lambda b: (0, 0)),
        ],
        out_specs=pl.BlockSpec((1, C, HW), lambda b: (b, 0, 0)),
        compiler_params=pltpu.CompilerParams(
            dimension_semantics=("parallel",),
            vmem_limit_bytes=48 << 20,
        ),
    )(x3, w, b_col)
    return out.reshape(B, C, H, W)
